# baseline (device time: 204803 ns/iter reference)
import jax
import jax.numpy as jnp
from jax import lax
from jax.experimental import pallas as pl
from jax.experimental.pallas import tpu as pltpu

HALF = 2048
D = 2048
NCHUNK = 8
CH = HALF // NCHUNK


def kernel(partial, gamma):
    p = partial.reshape(2 * HALF, D)
    g = gamma.reshape(1, D)

    def body(p_ref, g_ref, out_ref, comm_ref, stage_ref, sstage_ref,
             copy_sems, scopy_sems, send_sems, recv_sems):
        my_x = lax.axis_index("x")
        my_y = lax.axis_index("y")
        my_z = lax.axis_index("z")
        peer = (my_x, 1 - my_y, my_z)

        my_start = my_y * HALF
        peer_start = (1 - my_y) * HALF

        barrier_sem = pltpu.get_barrier_semaphore()
        pl.semaphore_signal(
            barrier_sem, inc=1,
            device_id=peer, device_id_type=pl.DeviceIdType.MESH,
        )
        pl.semaphore_wait(barrier_sem, 1)

        def send_copy(i, slot):
            return pltpu.make_async_copy(
                p_ref.at[pl.ds(peer_start + i * CH, CH), :],
                sstage_ref.at[slot],
                scopy_sems.at[slot],
            )

        def make_rdma(i, slot):
            return pltpu.make_async_remote_copy(
                src_ref=sstage_ref.at[slot],
                dst_ref=comm_ref.at[pl.ds(i * CH, CH), :],
                send_sem=send_sems.at[i],
                recv_sem=recv_sems.at[i],
                device_id=peer,
                device_id_type=pl.DeviceIdType.MESH,
            )

        send_copy(0, 0).start()
        send_copy(1, 1).start()
        rdmas = [None] * NCHUNK
        for i in range(NCHUNK):
            slot = i % 2
            if i >= 2:
                rdmas[i - 2].wait_send()
                send_copy(i, slot).start()
            send_copy(i, slot).wait()
            rdmas[i] = make_rdma(i, slot)
            rdmas[i].start()

        def local_copy(k, slot):
            return pltpu.make_async_copy(
                p_ref.at[pl.ds(my_start + k * CH, CH), :],
                stage_ref.at[slot],
                copy_sems.at[slot],
            )

        local_copy(0, 0).start()

        for k in range(NCHUNK):
            slot = k % 2
            if k + 1 < NCHUNK:
                local_copy(k + 1, (k + 1) % 2).start()
            local_copy(k, slot).wait()
            rdmas[k].wait_recv()
            rows = pl.ds(k * CH, CH)
            y = stage_ref[slot] + comm_ref[rows, :]
            ms = jnp.mean(y * y, axis=-1, keepdims=True) + 1e-6
            out_ref[rows, :] = y * lax.rsqrt(ms) * g_ref[:, :]

        rdmas[NCHUNK - 2].wait_send()
        rdmas[NCHUNK - 1].wait_send()

    return pl.pallas_call(
        body,
        out_shape=jax.ShapeDtypeStruct((HALF, D), jnp.float32),
        in_specs=[
            pl.BlockSpec(memory_space=pl.ANY),
            pl.BlockSpec(memory_space=pltpu.VMEM),
        ],
        out_specs=pl.BlockSpec(memory_space=pltpu.VMEM),
        scratch_shapes=[
            pltpu.VMEM((HALF, D), jnp.float32),
            pltpu.VMEM((2, CH, D), jnp.float32),
            pltpu.VMEM((2, CH, D), jnp.float32),
            pltpu.SemaphoreType.DMA((2,)),
            pltpu.SemaphoreType.DMA((2,)),
            pltpu.SemaphoreType.DMA((NCHUNK,)),
            pltpu.SemaphoreType.DMA((NCHUNK,)),
        ],
        compiler_params=pltpu.CompilerParams(
            collective_id=0,
            vmem_limit_bytes=60 * 1024 * 1024,
        ),
    )(p, g)


# device time: 115029 ns/iter; 1.7804x vs baseline; 1.7804x over previous
import jax
import jax.numpy as jnp
from jax import lax
from jax.experimental import pallas as pl
from jax.experimental.pallas import tpu as pltpu

HALF = 2048
D = 2048
NCHUNK = 8
CH = HALF // NCHUNK


def kernel(partial, gamma):
    p = partial.reshape(2 * HALF, D)
    g = gamma.reshape(1, D)

    def body(p_ref, g_ref, out_ref, comm_ref, stage_ref, sstage_ref, sbf_ref,
             copy_sems, scopy_sems, send_sems, recv_sems):
        my_x = lax.axis_index("x")
        my_y = lax.axis_index("y")
        my_z = lax.axis_index("z")
        peer = (my_x, 1 - my_y, my_z)

        my_start = my_y * HALF
        peer_start = (1 - my_y) * HALF

        barrier_sem = pltpu.get_barrier_semaphore()
        pl.semaphore_signal(
            barrier_sem, inc=1,
            device_id=peer, device_id_type=pl.DeviceIdType.MESH,
        )
        pl.semaphore_wait(barrier_sem, 1)

        def send_copy(i, slot):
            return pltpu.make_async_copy(
                p_ref.at[pl.ds(peer_start + i * CH, CH), :],
                sstage_ref.at[slot],
                scopy_sems.at[slot],
            )

        def make_rdma(i, slot):
            return pltpu.make_async_remote_copy(
                src_ref=sbf_ref.at[slot],
                dst_ref=comm_ref.at[pl.ds(i * CH, CH), :],
                send_sem=send_sems.at[i],
                recv_sem=recv_sems.at[i],
                device_id=peer,
                device_id_type=pl.DeviceIdType.MESH,
            )

        send_copy(0, 0).start()
        send_copy(1, 1).start()
        rdmas = [None] * NCHUNK
        for i in range(NCHUNK):
            slot = i % 2
            if i >= 2:
                rdmas[i - 2].wait_send()
            send_copy(i, slot).wait()
            sbf_ref[slot] = sstage_ref[slot].astype(jnp.bfloat16)
            if i + 2 < NCHUNK:
                send_copy(i + 2, slot).start()
            rdmas[i] = make_rdma(i, slot)
            rdmas[i].start()

        def local_copy(k, slot):
            return pltpu.make_async_copy(
                p_ref.at[pl.ds(my_start + k * CH, CH), :],
                stage_ref.at[slot],
                copy_sems.at[slot],
            )

        local_copy(0, 0).start()

        for k in range(NCHUNK):
            slot = k % 2
            if k + 1 < NCHUNK:
                local_copy(k + 1, (k + 1) % 2).start()
            local_copy(k, slot).wait()
            rdmas[k].wait_recv()
            rows = pl.ds(k * CH, CH)
            y = stage_ref[slot] + comm_ref[rows, :].astype(jnp.float32)
            ms = jnp.mean(y * y, axis=-1, keepdims=True) + 1e-6
            out_ref[rows, :] = y * lax.rsqrt(ms) * g_ref[:, :]

        rdmas[NCHUNK - 2].wait_send()
        rdmas[NCHUNK - 1].wait_send()

    return pl.pallas_call(
        body,
        out_shape=jax.ShapeDtypeStruct((HALF, D), jnp.float32),
        in_specs=[
            pl.BlockSpec(memory_space=pl.ANY),
            pl.BlockSpec(memory_space=pltpu.VMEM),
        ],
        out_specs=pl.BlockSpec(memory_space=pltpu.VMEM),
        scratch_shapes=[
            pltpu.VMEM((HALF, D), jnp.bfloat16),
            pltpu.VMEM((2, CH, D), jnp.float32),
            pltpu.VMEM((2, CH, D), jnp.float32),
            pltpu.VMEM((2, CH, D), jnp.bfloat16),
            pltpu.SemaphoreType.DMA((2,)),
            pltpu.SemaphoreType.DMA((2,)),
            pltpu.SemaphoreType.DMA((NCHUNK,)),
            pltpu.SemaphoreType.DMA((NCHUNK,)),
        ],
        compiler_params=pltpu.CompilerParams(
            collective_id=0,
            vmem_limit_bytes=60 * 1024 * 1024,
        ),
    )(p, g)


# device time: 113472 ns/iter; 1.8049x vs baseline; 1.0137x over previous
import jax
import jax.numpy as jnp
from jax import lax
from jax.experimental import pallas as pl
from jax.experimental.pallas import tpu as pltpu

HALF = 2048
D = 2048
NCHUNK = 16
CH = HALF // NCHUNK


def kernel(partial, gamma):
    p = partial.reshape(2 * HALF, D)
    g = gamma.reshape(1, D)

    def body(p_ref, g_ref, out_ref, comm_ref, stage_ref, sstage_ref, sbf_ref,
             res_ref, copy_sems, scopy_sems, ocopy_sems, send_sems, recv_sems):
        my_x = lax.axis_index("x")
        my_y = lax.axis_index("y")
        my_z = lax.axis_index("z")
        peer = (my_x, 1 - my_y, my_z)

        my_start = my_y * HALF
        peer_start = (1 - my_y) * HALF

        barrier_sem = pltpu.get_barrier_semaphore()
        pl.semaphore_signal(
            barrier_sem, inc=1,
            device_id=peer, device_id_type=pl.DeviceIdType.MESH,
        )
        pl.semaphore_wait(barrier_sem, 1)

        def send_copy(i, slot):
            return pltpu.make_async_copy(
                p_ref.at[pl.ds(peer_start + i * CH, CH), :],
                sstage_ref.at[slot],
                scopy_sems.at[slot],
            )

        def make_rdma(i, slot):
            return pltpu.make_async_remote_copy(
                src_ref=sbf_ref.at[slot],
                dst_ref=comm_ref.at[pl.ds(i * CH, CH), :],
                send_sem=send_sems.at[i],
                recv_sem=recv_sems.at[i],
                device_id=peer,
                device_id_type=pl.DeviceIdType.MESH,
            )

        send_copy(0, 0).start()
        send_copy(1, 1).start()
        rdmas = [None] * NCHUNK
        for i in range(NCHUNK):
            slot = i % 2
            if i >= 2:
                rdmas[i - 2].wait_send()
            send_copy(i, slot).wait()
            sbf_ref[slot] = sstage_ref[slot].astype(jnp.bfloat16)
            if i + 2 < NCHUNK:
                send_copy(i + 2, slot).start()
            rdmas[i] = make_rdma(i, slot)
            rdmas[i].start()

        def local_copy(k, slot):
            return pltpu.make_async_copy(
                p_ref.at[pl.ds(my_start + k * CH, CH), :],
                stage_ref.at[slot],
                copy_sems.at[slot],
            )

        def out_copy(k, slot):
            return pltpu.make_async_copy(
                res_ref.at[slot],
                out_ref.at[pl.ds(k * CH, CH), :],
                ocopy_sems.at[slot],
            )

        local_copy(0, 0).start()

        for k in range(NCHUNK):
            slot = k % 2
            if k + 1 < NCHUNK:
                local_copy(k + 1, (k + 1) % 2).start()
            local_copy(k, slot).wait()
            rdmas[k].wait_recv()
            if k >= 2:
                out_copy(k - 2, slot).wait()
            rows = pl.ds(k * CH, CH)
            y = stage_ref[slot] + comm_ref[rows, :].astype(jnp.float32)
            ms = jnp.mean(y * y, axis=-1, keepdims=True) + 1e-6
            res_ref[slot] = y * lax.rsqrt(ms) * g_ref[:, :]
            out_copy(k, slot).start()

        out_copy(NCHUNK - 2, (NCHUNK - 2) % 2).wait()
        out_copy(NCHUNK - 1, (NCHUNK - 1) % 2).wait()
        rdmas[NCHUNK - 2].wait_send()
        rdmas[NCHUNK - 1].wait_send()

    return pl.pallas_call(
        body,
        out_shape=jax.ShapeDtypeStruct((HALF, D), jnp.float32),
        in_specs=[
            pl.BlockSpec(memory_space=pl.ANY),
            pl.BlockSpec(memory_space=pltpu.VMEM),
        ],
        out_specs=pl.BlockSpec(memory_space=pl.ANY),
        scratch_shapes=[
            pltpu.VMEM((HALF, D), jnp.bfloat16),
            pltpu.VMEM((2, CH, D), jnp.float32),
            pltpu.VMEM((2, CH, D), jnp.float32),
            pltpu.VMEM((2, CH, D), jnp.bfloat16),
            pltpu.VMEM((2, CH, D), jnp.float32),
            pltpu.SemaphoreType.DMA((2,)),
            pltpu.SemaphoreType.DMA((2,)),
            pltpu.SemaphoreType.DMA((2,)),
            pltpu.SemaphoreType.DMA((NCHUNK,)),
            pltpu.SemaphoreType.DMA((NCHUNK,)),
        ],
        compiler_params=pltpu.CompilerParams(
            collective_id=0,
            vmem_limit_bytes=60 * 1024 * 1024,
        ),
    )(p, g)


# device time: 101970 ns/iter; 2.0085x vs baseline; 1.1128x over previous
import jax
import jax.numpy as jnp
from jax import lax
from jax.experimental import pallas as pl
from jax.experimental.pallas import tpu as pltpu

HALF = 2048
D = 2048
Q = HALF // 2
NCHUNK = 8
CH = Q // NCHUNK


def kernel(partial, gamma):
    p = partial.reshape(2 * HALF, D)
    g = gamma.reshape(1, D)

    def body(p_ref, g_ref, out_ref, comm1_ref, comm2_ref, stage_ref,
             sstage_ref, sbf_ref, res_ref, sbf2_ref, res2_ref,
             copy_sems, scopy_sems, ocopy_sems, o2copy_sems,
             send1_sems, recv1_sems, send2_sems, recv2_sems):
        my_x = lax.axis_index("x")
        my_y = lax.axis_index("y")
        my_z = lax.axis_index("z")
        ypeer = (my_x, 1 - my_y, my_z)
        xpeer = (1 - my_x, my_y, my_z)

        my_start = my_y * HALF
        peer_start = (1 - my_y) * HALF
        blk = my_x * Q
        oblk = (1 - my_x) * Q

        barrier_sem = pltpu.get_barrier_semaphore()
        for nbr in (ypeer, xpeer):
            pl.semaphore_signal(
                barrier_sem, inc=1,
                device_id=nbr, device_id_type=pl.DeviceIdType.MESH,
            )
        pl.semaphore_wait(barrier_sem, 2)

        def send_copy(i, slot):
            return pltpu.make_async_copy(
                p_ref.at[pl.ds(peer_start + blk + i * CH, CH), :],
                sstage_ref.at[slot],
                scopy_sems.at[slot],
            )

        def make_rdma1(i, slot):
            return pltpu.make_async_remote_copy(
                src_ref=sbf_ref.at[slot],
                dst_ref=comm1_ref.at[pl.ds(i * CH, CH), :],
                send_sem=send1_sems.at[i],
                recv_sem=recv1_sems.at[i],
                device_id=ypeer,
                device_id_type=pl.DeviceIdType.MESH,
            )

        send_copy(0, 0).start()
        send_copy(1, 1).start()
        rdmas1 = [None] * NCHUNK
        for i in range(NCHUNK):
            slot = i % 2
            if i >= 2:
                rdmas1[i - 2].wait_send()
            send_copy(i, slot).wait()
            sbf_ref[slot] = sstage_ref[slot].astype(jnp.bfloat16)
            if i + 2 < NCHUNK:
                send_copy(i + 2, slot).start()
            rdmas1[i] = make_rdma1(i, slot)
            rdmas1[i].start()

        def local_copy(k, slot):
            return pltpu.make_async_copy(
                p_ref.at[pl.ds(my_start + blk + k * CH, CH), :],
                stage_ref.at[slot],
                copy_sems.at[slot],
            )

        def out_copy(k, slot):
            return pltpu.make_async_copy(
                res_ref.at[slot],
                out_ref.at[pl.ds(blk + k * CH, CH), :],
                ocopy_sems.at[slot],
            )

        def make_rdma2(k, slot):
            return pltpu.make_async_remote_copy(
                src_ref=sbf2_ref.at[slot],
                dst_ref=comm2_ref.at[pl.ds(k * CH, CH), :],
                send_sem=send2_sems.at[k],
                recv_sem=recv2_sems.at[k],
                device_id=xpeer,
                device_id_type=pl.DeviceIdType.MESH,
            )

        def out2_copy(j, slot):
            return pltpu.make_async_copy(
                res2_ref.at[slot],
                out_ref.at[pl.ds(oblk + j * CH, CH), :],
                o2copy_sems.at[slot],
            )

        rdmas2 = [None] * NCHUNK

        def process2(j):
            s2 = j % 2
            rdmas2[j].wait_recv()
            if j >= 2:
                out2_copy(j - 2, s2).wait()
            res2_ref[s2] = comm2_ref[pl.ds(j * CH, CH), :].astype(jnp.float32)
            out2_copy(j, s2).start()

        local_copy(0, 0).start()
        for k in range(NCHUNK):
            slot = k % 2
            if k + 1 < NCHUNK:
                local_copy(k + 1, (k + 1) % 2).start()
            local_copy(k, slot).wait()
            rdmas1[k].wait_recv()
            if k >= 2:
                out_copy(k - 2, slot).wait()
            rows = pl.ds(k * CH, CH)
            y = stage_ref[slot] + comm1_ref[rows, :].astype(jnp.float32)
            ms = jnp.mean(y * y, axis=-1, keepdims=True) + 1e-6
            r = y * lax.rsqrt(ms) * g_ref[:, :]
            res_ref[slot] = r
            out_copy(k, slot).start()
            if k >= 2:
                rdmas2[k - 2].wait_send()
            sbf2_ref[slot] = r.astype(jnp.bfloat16)
            rdmas2[k] = make_rdma2(k, slot)
            rdmas2[k].start()
            if k >= 2:
                process2(k - 2)

        process2(NCHUNK - 2)
        process2(NCHUNK - 1)
        out_copy(NCHUNK - 2, (NCHUNK - 2) % 2).wait()
        out_copy(NCHUNK - 1, (NCHUNK - 1) % 2).wait()
        out2_copy(NCHUNK - 2, (NCHUNK - 2) % 2).wait()
        out2_copy(NCHUNK - 1, (NCHUNK - 1) % 2).wait()
        rdmas1[NCHUNK - 2].wait_send()
        rdmas1[NCHUNK - 1].wait_send()
        rdmas2[NCHUNK - 2].wait_send()
        rdmas2[NCHUNK - 1].wait_send()

    return pl.pallas_call(
        body,
        out_shape=jax.ShapeDtypeStruct((HALF, D), jnp.float32),
        in_specs=[
            pl.BlockSpec(memory_space=pl.ANY),
            pl.BlockSpec(memory_space=pltpu.VMEM),
        ],
        out_specs=pl.BlockSpec(memory_space=pl.ANY),
        scratch_shapes=[
            pltpu.VMEM((Q, D), jnp.bfloat16),
            pltpu.VMEM((Q, D), jnp.bfloat16),
            pltpu.VMEM((2, CH, D), jnp.float32),
            pltpu.VMEM((2, CH, D), jnp.float32),
            pltpu.VMEM((2, CH, D), jnp.bfloat16),
            pltpu.VMEM((2, CH, D), jnp.float32),
            pltpu.VMEM((2, CH, D), jnp.bfloat16),
            pltpu.VMEM((2, CH, D), jnp.float32),
            pltpu.SemaphoreType.DMA((2,)),
            pltpu.SemaphoreType.DMA((2,)),
            pltpu.SemaphoreType.DMA((2,)),
            pltpu.SemaphoreType.DMA((2,)),
            pltpu.SemaphoreType.DMA((NCHUNK,)),
            pltpu.SemaphoreType.DMA((NCHUNK,)),
            pltpu.SemaphoreType.DMA((NCHUNK,)),
            pltpu.SemaphoreType.DMA((NCHUNK,)),
        ],
        compiler_params=pltpu.CompilerParams(
            collective_id=0,
            vmem_limit_bytes=60 * 1024 * 1024,
        ),
    )(p, g)


# device time: 73186 ns/iter; 2.7984x vs baseline; 1.3933x over previous
import jax
import jax.numpy as jnp
from jax import lax
from jax.experimental import pallas as pl
from jax.experimental.pallas import tpu as pltpu

HALF = 2048
D = 2048
Q = HALF // 2
NCHUNK = 8
CH = Q // NCHUNK


def kernel(partial, gamma):
    p = partial.reshape(2 * HALF, D)
    g = gamma.reshape(1, D)

    def body(p_ref, g_ref, out_ref, comm1_ref, comm2_ref, stage_ref,
             sstage_ref, sbf_ref, res_ref, sbf2_ref, res2_ref,
             copy_sems, scopy_sems, ocopy_sems, o2copy_sems,
             send1_sems, recv1_sems, send2_sems, recv2_sems):
        my_x = lax.axis_index("x")
        my_y = lax.axis_index("y")
        my_z = lax.axis_index("z")
        ypeer = (my_x, 1 - my_y, my_z)
        xpeer = (1 - my_x, my_y, my_z)

        my_start = my_y * HALF
        peer_start = (1 - my_y) * HALF
        blk = my_x * Q
        oblk = (1 - my_x) * Q

        barrier_sem = pltpu.get_barrier_semaphore()
        for nbr in (ypeer, xpeer):
            pl.semaphore_signal(
                barrier_sem, inc=1,
                device_id=nbr, device_id_type=pl.DeviceIdType.MESH,
            )
        pl.semaphore_wait(barrier_sem, 2)

        def send_copy(i):
            return pltpu.make_async_copy(
                p_ref.at[pl.ds(peer_start + blk + i * CH, CH), :],
                sstage_ref.at[i],
                scopy_sems.at[i],
            )

        def make_rdma1(i):
            return pltpu.make_async_remote_copy(
                src_ref=sbf_ref.at[i],
                dst_ref=comm1_ref.at[pl.ds(i * CH, CH), :],
                send_sem=send1_sems.at[i],
                recv_sem=recv1_sems.at[i],
                device_id=ypeer,
                device_id_type=pl.DeviceIdType.MESH,
            )

        for i in range(NCHUNK):
            send_copy(i).start()
        rdmas1 = [None] * NCHUNK
        for i in range(NCHUNK):
            send_copy(i).wait()
            sbf_ref[i] = sstage_ref[i].astype(jnp.bfloat16)
            rdmas1[i] = make_rdma1(i)
            rdmas1[i].start()

        def local_copy(k, slot):
            return pltpu.make_async_copy(
                p_ref.at[pl.ds(my_start + blk + k * CH, CH), :],
                stage_ref.at[slot],
                copy_sems.at[slot],
            )

        def out_copy(k, slot):
            return pltpu.make_async_copy(
                res_ref.at[slot],
                out_ref.at[pl.ds(blk + k * CH, CH), :],
                ocopy_sems.at[slot],
            )

        def make_rdma2(k):
            return pltpu.make_async_remote_copy(
                src_ref=sbf2_ref.at[k],
                dst_ref=comm2_ref.at[pl.ds(k * CH, CH), :],
                send_sem=send2_sems.at[k],
                recv_sem=recv2_sems.at[k],
                device_id=xpeer,
                device_id_type=pl.DeviceIdType.MESH,
            )

        def out2_copy(j, slot):
            return pltpu.make_async_copy(
                res2_ref.at[slot],
                out_ref.at[pl.ds(oblk + j * CH, CH), :],
                o2copy_sems.at[slot],
            )

        rdmas2 = [None] * NCHUNK

        def process2(j):
            s2 = j % 2
            rdmas2[j].wait_recv()
            if j >= 2:
                out2_copy(j - 2, s2).wait()
            res2_ref[s2] = comm2_ref[pl.ds(j * CH, CH), :].astype(jnp.float32)
            out2_copy(j, s2).start()

        local_copy(0, 0).start()
        for k in range(NCHUNK):
            slot = k % 2
            if k + 1 < NCHUNK:
                local_copy(k + 1, (k + 1) % 2).start()
            local_copy(k, slot).wait()
            rdmas1[k].wait_recv()
            if k >= 2:
                out_copy(k - 2, slot).wait()
            rows = pl.ds(k * CH, CH)
            y = stage_ref[slot] + comm1_ref[rows, :].astype(jnp.float32)
            ms = jnp.mean(y * y, axis=-1, keepdims=True) + 1e-6
            r = y * lax.rsqrt(ms) * g_ref[:, :]
            res_ref[slot] = r
            out_copy(k, slot).start()
            sbf2_ref[k] = r.astype(jnp.bfloat16)
            rdmas2[k] = make_rdma2(k)
            rdmas2[k].start()
            if k >= 2:
                process2(k - 2)

        process2(NCHUNK - 2)
        process2(NCHUNK - 1)
        out_copy(NCHUNK - 2, (NCHUNK - 2) % 2).wait()
        out_copy(NCHUNK - 1, (NCHUNK - 1) % 2).wait()
        out2_copy(NCHUNK - 2, (NCHUNK - 2) % 2).wait()
        out2_copy(NCHUNK - 1, (NCHUNK - 1) % 2).wait()
        for i in range(NCHUNK):
            rdmas1[i].wait_send()
            rdmas2[i].wait_send()

    return pl.pallas_call(
        body,
        out_shape=jax.ShapeDtypeStruct((HALF, D), jnp.float32),
        in_specs=[
            pl.BlockSpec(memory_space=pl.ANY),
            pl.BlockSpec(memory_space=pltpu.VMEM),
        ],
        out_specs=pl.BlockSpec(memory_space=pl.ANY),
        scratch_shapes=[
            pltpu.VMEM((Q, D), jnp.bfloat16),
            pltpu.VMEM((Q, D), jnp.bfloat16),
            pltpu.VMEM((2, CH, D), jnp.float32),
            pltpu.VMEM((NCHUNK, CH, D), jnp.float32),
            pltpu.VMEM((NCHUNK, CH, D), jnp.bfloat16),
            pltpu.VMEM((2, CH, D), jnp.float32),
            pltpu.VMEM((NCHUNK, CH, D), jnp.bfloat16),
            pltpu.VMEM((2, CH, D), jnp.float32),
            pltpu.SemaphoreType.DMA((2,)),
            pltpu.SemaphoreType.DMA((NCHUNK,)),
            pltpu.SemaphoreType.DMA((2,)),
            pltpu.SemaphoreType.DMA((2,)),
            pltpu.SemaphoreType.DMA((NCHUNK,)),
            pltpu.SemaphoreType.DMA((NCHUNK,)),
            pltpu.SemaphoreType.DMA((NCHUNK,)),
            pltpu.SemaphoreType.DMA((NCHUNK,)),
        ],
        compiler_params=pltpu.CompilerParams(
            collective_id=0,
            vmem_limit_bytes=60 * 1024 * 1024,
        ),
    )(p, g)


# device time: 70108 ns/iter; 2.9213x vs baseline; 1.0439x over previous
import jax
import jax.numpy as jnp
from jax import lax
from jax.experimental import pallas as pl
from jax.experimental.pallas import tpu as pltpu

HALF = 2048
D = 2048
Q = HALF // 2
NCHUNK = 16
CH = Q // NCHUNK


def kernel(partial, gamma):
    p = partial.reshape(2 * HALF, D)
    g = gamma.reshape(1, D)

    def body(p_ref, g_ref, out_ref, comm1_ref, comm2_ref, stage_ref,
             sstage_ref, sbf_ref, res_ref, sbf2_ref, res2_ref,
             copy_sems, scopy_sems, ocopy_sems, o2copy_sems,
             send1_sems, recv1_sems, send2_sems, recv2_sems):
        my_x = lax.axis_index("x")
        my_y = lax.axis_index("y")
        my_z = lax.axis_index("z")
        ypeer = (my_x, 1 - my_y, my_z)
        xpeer = (1 - my_x, my_y, my_z)

        my_start = my_y * HALF
        peer_start = (1 - my_y) * HALF
        blk = my_x * Q
        oblk = (1 - my_x) * Q

        barrier_sem = pltpu.get_barrier_semaphore()
        for nbr in (ypeer, xpeer):
            pl.semaphore_signal(
                barrier_sem, inc=1,
                device_id=nbr, device_id_type=pl.DeviceIdType.MESH,
            )
        pl.semaphore_wait(barrier_sem, 2)

        def send_copy(i):
            return pltpu.make_async_copy(
                p_ref.at[pl.ds(peer_start + blk + i * CH, CH), :],
                sstage_ref.at[i],
                scopy_sems.at[i],
            )

        def make_rdma1(i):
            return pltpu.make_async_remote_copy(
                src_ref=sbf_ref.at[i],
                dst_ref=comm1_ref.at[pl.ds(i * CH, CH), :],
                send_sem=send1_sems.at[i],
                recv_sem=recv1_sems.at[i],
                device_id=ypeer,
                device_id_type=pl.DeviceIdType.MESH,
            )

        for i in range(NCHUNK):
            send_copy(i).start()
        rdmas1 = [None] * NCHUNK
        for i in range(NCHUNK):
            send_copy(i).wait()
            sbf_ref[i] = sstage_ref[i].astype(jnp.bfloat16)
            rdmas1[i] = make_rdma1(i)
            rdmas1[i].start()

        def local_copy(k, slot):
            return pltpu.make_async_copy(
                p_ref.at[pl.ds(my_start + blk + k * CH, CH), :],
                stage_ref.at[slot],
                copy_sems.at[slot],
            )

        def out_copy(k, slot):
            return pltpu.make_async_copy(
                res_ref.at[slot],
                out_ref.at[pl.ds(blk + k * CH, CH), :],
                ocopy_sems.at[slot],
            )

        def make_rdma2(k):
            return pltpu.make_async_remote_copy(
                src_ref=sbf2_ref.at[k],
                dst_ref=comm2_ref.at[pl.ds(k * CH, CH), :],
                send_sem=send2_sems.at[k],
                recv_sem=recv2_sems.at[k],
                device_id=xpeer,
                device_id_type=pl.DeviceIdType.MESH,
            )

        def out2_copy(j, slot):
            return pltpu.make_async_copy(
                res2_ref.at[slot],
                out_ref.at[pl.ds(oblk + j * CH, CH), :],
                o2copy_sems.at[slot],
            )

        rdmas2 = [None] * NCHUNK

        def process2(j):
            s2 = j % 2
            rdmas2[j].wait_recv()
            if j >= 2:
                out2_copy(j - 2, s2).wait()
            res2_ref[s2] = comm2_ref[pl.ds(j * CH, CH), :].astype(jnp.float32)
            out2_copy(j, s2).start()

        local_copy(0, 0).start()
        for k in range(NCHUNK):
            slot = k % 2
            if k + 1 < NCHUNK:
                local_copy(k + 1, (k + 1) % 2).start()
            local_copy(k, slot).wait()
            rdmas1[k].wait_recv()
            if k >= 2:
                out_copy(k - 2, slot).wait()
            rows = pl.ds(k * CH, CH)
            y = stage_ref[slot] + comm1_ref[rows, :].astype(jnp.float32)
            ms = jnp.mean(y * y, axis=-1, keepdims=True) + 1e-6
            r = y * lax.rsqrt(ms) * g_ref[:, :]
            res_ref[slot] = r
            out_copy(k, slot).start()
            sbf2_ref[k] = r.astype(jnp.bfloat16)
            rdmas2[k] = make_rdma2(k)
            rdmas2[k].start()
            if k >= 2:
                process2(k - 2)

        process2(NCHUNK - 2)
        process2(NCHUNK - 1)
        out_copy(NCHUNK - 2, (NCHUNK - 2) % 2).wait()
        out_copy(NCHUNK - 1, (NCHUNK - 1) % 2).wait()
        out2_copy(NCHUNK - 2, (NCHUNK - 2) % 2).wait()
        out2_copy(NCHUNK - 1, (NCHUNK - 1) % 2).wait()
        for i in range(NCHUNK):
            rdmas1[i].wait_send()
            rdmas2[i].wait_send()

    return pl.pallas_call(
        body,
        out_shape=jax.ShapeDtypeStruct((HALF, D), jnp.float32),
        in_specs=[
            pl.BlockSpec(memory_space=pl.ANY),
            pl.BlockSpec(memory_space=pltpu.VMEM),
        ],
        out_specs=pl.BlockSpec(memory_space=pl.ANY),
        scratch_shapes=[
            pltpu.VMEM((Q, D), jnp.bfloat16),
            pltpu.VMEM((Q, D), jnp.bfloat16),
            pltpu.VMEM((2, CH, D), jnp.float32),
            pltpu.VMEM((NCHUNK, CH, D), jnp.float32),
            pltpu.VMEM((NCHUNK, CH, D), jnp.bfloat16),
            pltpu.VMEM((2, CH, D), jnp.float32),
            pltpu.VMEM((NCHUNK, CH, D), jnp.bfloat16),
            pltpu.VMEM((2, CH, D), jnp.float32),
            pltpu.SemaphoreType.DMA((2,)),
            pltpu.SemaphoreType.DMA((NCHUNK,)),
            pltpu.SemaphoreType.DMA((2,)),
            pltpu.SemaphoreType.DMA((2,)),
            pltpu.SemaphoreType.DMA((NCHUNK,)),
            pltpu.SemaphoreType.DMA((NCHUNK,)),
            pltpu.SemaphoreType.DMA((NCHUNK,)),
            pltpu.SemaphoreType.DMA((NCHUNK,)),
        ],
        compiler_params=pltpu.CompilerParams(
            collective_id=0,
            vmem_limit_bytes=60 * 1024 * 1024,
        ),
    )(p, g)
